# baseline (speedup 1.0000x reference)
import jax
import jax.numpy as jnp
from jax.experimental import pallas as pl
from jax.experimental.pallas import tpu as pltpu

_MIN_NORM = 1e-15
_BALL_EPS = 1e-5
_NBUF = 6


def _rownorm(v):
    return jnp.maximum(jnp.sqrt(jnp.sum(v * v, axis=-1, keepdims=True)), _MIN_NORM)


def _make_body(bm, nblocks):
    def _fused_body(x_ref, adj_hbm, w_ref, b_ref, out_ref, abuf, s_ref, sems):
        i = pl.program_id(0)
        maxnorm = 1.0 - _BALL_EPS

        def _fetch(blk):
            pltpu.make_async_copy(
                adj_hbm.at[pl.ds(blk * bm, bm), :],
                abuf.at[blk % _NBUF],
                sems.at[blk % _NBUF],
            ).start()

        @pl.when(i == 0)
        def _prologue():
            for k in range(min(_NBUF, nblocks)):
                _fetch(k)
            xv = x_ref[...]
            norm = _rownorm(xv)
            n2 = jnp.minimum(norm, maxnorm)
            at = 0.5 * jnp.log((1.0 + n2) / (1.0 - n2))
            xt = xv * (at / norm)
            support = jax.lax.dot_general(
                xt, w_ref[...], (((1,), (0,)), ((), ())),
                preferred_element_type=jnp.float32,
            ) + b_ref[...]
            s_ref[...] = support.astype(jnp.bfloat16)

        @pl.when((i > 0) & (i + _NBUF - 1 < nblocks))
        def _prefetch():
            _fetch(i + _NBUF - 1)

        pltpu.make_async_copy(
            adj_hbm.at[pl.ds(i * bm, bm), :],
            abuf.at[i % _NBUF],
            sems.at[i % _NBUF],
        ).wait()
        acc = jax.lax.dot_general(
            abuf[i % _NBUF].astype(jnp.bfloat16), s_ref[...],
            (((1,), (0,)), ((), ())),
            preferred_element_type=jnp.float32,
        )
        norm = _rownorm(acc)
        t = jnp.tanh(norm)
        out_ref[...] = acc * (jnp.minimum(t, maxnorm) / norm)

    return _fused_body


def kernel(x, adj, weight, bias):
    n, d_in = x.shape
    d_out = weight.shape[1]
    bias2 = bias.reshape(1, d_out).astype(jnp.float32)

    bm = 200 if n % 200 == 0 else n
    nblocks = n // bm
    out = pl.pallas_call(
        _make_body(bm, nblocks),
        grid=(nblocks,),
        in_specs=[
            pl.BlockSpec((n, d_in), lambda i: (0, 0)),
            pl.BlockSpec(memory_space=pl.ANY),
            pl.BlockSpec((d_in, d_out), lambda i: (0, 0)),
            pl.BlockSpec((1, d_out), lambda i: (0, 0)),
        ],
        out_specs=pl.BlockSpec((bm, d_out), lambda i: (i, 0)),
        out_shape=jax.ShapeDtypeStruct((n, d_out), jnp.float32),
        scratch_shapes=[
            pltpu.VMEM((_NBUF, bm, n), jnp.float32),
            pltpu.VMEM((n, d_out), jnp.bfloat16),
            pltpu.SemaphoreType.DMA((_NBUF,)),
        ],
        compiler_params=pltpu.CompilerParams(
            dimension_semantics=("arbitrary",)),
    )(x, adj, weight, bias2)
    return out


# speedup vs baseline: 1.0143x; 1.0143x over previous
import jax
import jax.numpy as jnp
from jax.experimental import pallas as pl
from jax.experimental.pallas import tpu as pltpu

_MIN_NORM = 1e-15
_BALL_EPS = 1e-5
_NBUF = 5


def _rownorm(v):
    return jnp.maximum(jnp.sqrt(jnp.sum(v * v, axis=-1, keepdims=True)), _MIN_NORM)


def _make_body(bm, nblocks):
    def _fused_body(x_ref, adj_hbm, w_ref, b_ref, out_ref, abuf, s_ref, sems):
        i = pl.program_id(0)
        maxnorm = 1.0 - _BALL_EPS

        def _fetch(blk):
            pltpu.make_async_copy(
                adj_hbm.at[pl.ds(blk * bm, bm), :],
                abuf.at[blk % _NBUF],
                sems.at[blk % _NBUF],
            ).start()

        @pl.when(i == 0)
        def _prologue():
            for k in range(min(_NBUF, nblocks)):
                _fetch(k)
            xv = x_ref[...]
            norm = _rownorm(xv)
            n2 = jnp.minimum(norm, maxnorm)
            at = 0.5 * jnp.log((1.0 + n2) / (1.0 - n2))
            xt = xv * (at / norm)
            support = jax.lax.dot_general(
                xt, w_ref[...], (((1,), (0,)), ((), ())),
                preferred_element_type=jnp.float32,
            ) + b_ref[...]
            s_ref[...] = support.astype(jnp.bfloat16)

        @pl.when((i > 0) & (i + _NBUF - 1 < nblocks))
        def _prefetch():
            _fetch(i + _NBUF - 1)

        pltpu.make_async_copy(
            adj_hbm.at[pl.ds(i * bm, bm), :],
            abuf.at[i % _NBUF],
            sems.at[i % _NBUF],
        ).wait()
        acc = jax.lax.dot_general(
            abuf[i % _NBUF].astype(jnp.bfloat16), s_ref[...],
            (((1,), (0,)), ((), ())),
            preferred_element_type=jnp.float32,
        )
        norm = _rownorm(acc)
        t = jnp.tanh(norm)
        out_ref[...] = acc * (jnp.minimum(t, maxnorm) / norm)

    return _fused_body


def kernel(x, adj, weight, bias):
    n, d_in = x.shape
    d_out = weight.shape[1]
    bias2 = bias.reshape(1, d_out).astype(jnp.float32)

    bm = 200 if n % 200 == 0 else n
    nblocks = n // bm
    out = pl.pallas_call(
        _make_body(bm, nblocks),
        grid=(nblocks,),
        in_specs=[
            pl.BlockSpec((n, d_in), lambda i: (0, 0)),
            pl.BlockSpec(memory_space=pl.ANY),
            pl.BlockSpec((d_in, d_out), lambda i: (0, 0)),
            pl.BlockSpec((1, d_out), lambda i: (0, 0)),
        ],
        out_specs=pl.BlockSpec((bm, d_out), lambda i: (i, 0)),
        out_shape=jax.ShapeDtypeStruct((n, d_out), jnp.float32),
        scratch_shapes=[
            pltpu.VMEM((_NBUF, bm, n), jnp.float32),
            pltpu.VMEM((n, d_out), jnp.bfloat16),
            pltpu.SemaphoreType.DMA((_NBUF,)),
        ],
        compiler_params=pltpu.CompilerParams(
            dimension_semantics=("arbitrary",)),
    )(x, adj, weight, bias2)
    return out


# speedup vs baseline: 1.0410x; 1.0263x over previous
import jax
import jax.numpy as jnp
from jax.experimental import pallas as pl
from jax.experimental.pallas import tpu as pltpu

_MIN_NORM = 1e-15
_BALL_EPS = 1e-5
_NBUF = 8


def _rownorm(v):
    return jnp.maximum(jnp.sqrt(jnp.sum(v * v, axis=-1, keepdims=True)), _MIN_NORM)


def _make_body(bm, nblocks):
    def _fused_body(x_ref, adj_hbm, w_ref, b_ref, out_ref, abuf, s_ref, sems):
        i = pl.program_id(0)
        maxnorm = 1.0 - _BALL_EPS

        def _fetch(blk):
            pltpu.make_async_copy(
                adj_hbm.at[pl.ds(blk * bm, bm), :],
                abuf.at[blk % _NBUF],
                sems.at[blk % _NBUF],
            ).start()

        @pl.when(i == 0)
        def _prologue():
            for k in range(min(_NBUF, nblocks)):
                _fetch(k)
            xv = x_ref[...]
            norm = _rownorm(xv)
            n2 = jnp.minimum(norm, maxnorm)
            at = 0.5 * jnp.log((1.0 + n2) / (1.0 - n2))
            xt = xv * (at / norm)
            support = jax.lax.dot_general(
                xt, w_ref[...], (((1,), (0,)), ((), ())),
                preferred_element_type=jnp.float32,
            ) + b_ref[...]
            s_ref[...] = support.astype(jnp.bfloat16)

        @pl.when((i > 0) & (i + _NBUF - 1 < nblocks))
        def _prefetch():
            _fetch(i + _NBUF - 1)

        pltpu.make_async_copy(
            adj_hbm.at[pl.ds(i * bm, bm), :],
            abuf.at[i % _NBUF],
            sems.at[i % _NBUF],
        ).wait()
        acc = jax.lax.dot_general(
            abuf[i % _NBUF].astype(jnp.bfloat16), s_ref[...],
            (((1,), (0,)), ((), ())),
            preferred_element_type=jnp.float32,
        )
        norm = _rownorm(acc)
        t = jnp.tanh(norm)
        out_ref[...] = acc * (jnp.minimum(t, maxnorm) / norm)

    return _fused_body


def kernel(x, adj, weight, bias):
    n, d_in = x.shape
    d_out = weight.shape[1]
    bias2 = bias.reshape(1, d_out).astype(jnp.float32)

    bm = 80 if n % 80 == 0 else n
    nblocks = n // bm
    out = pl.pallas_call(
        _make_body(bm, nblocks),
        grid=(nblocks,),
        in_specs=[
            pl.BlockSpec((n, d_in), lambda i: (0, 0)),
            pl.BlockSpec(memory_space=pl.ANY),
            pl.BlockSpec((d_in, d_out), lambda i: (0, 0)),
            pl.BlockSpec((1, d_out), lambda i: (0, 0)),
        ],
        out_specs=pl.BlockSpec((bm, d_out), lambda i: (i, 0)),
        out_shape=jax.ShapeDtypeStruct((n, d_out), jnp.float32),
        scratch_shapes=[
            pltpu.VMEM((_NBUF, bm, n), jnp.float32),
            pltpu.VMEM((n, d_out), jnp.bfloat16),
            pltpu.SemaphoreType.DMA((_NBUF,)),
        ],
        compiler_params=pltpu.CompilerParams(
            dimension_semantics=("arbitrary",)),
    )(x, adj, weight, bias2)
    return out
